# baseline (speedup 1.0000x reference)
import jax
import jax.numpy as jnp
from jax import lax
from jax.experimental import pallas as pl
from jax.experimental.pallas import tpu as pltpu
from jax.experimental.pallas import tpu_sc as plsc

N = 10000
E = 320000
D = 128
O = 128
WW = 16

NC = 2
NS = 16
NW = NC * NS
CH = 80
NCHUNK = E // CH
CPT = NCHUNK // NW
G = 5
NP = 10240
ROWS_PER_TILE = NP // NS


def _proj_body(x_ref, qwt_ref, qb_ref, o_ref):
    z = jnp.dot(x_ref[...], qwt_ref[...], preferred_element_type=jnp.float32)
    o_ref[...] = jnp.maximum(z + qb_ref[...], 0.0).astype(jnp.bfloat16)


def _project(h_src, qwt, qb):
    blk = 2000
    return pl.pallas_call(
        _proj_body,
        grid=(N // blk,),
        in_specs=[
            pl.BlockSpec((blk, D), lambda i: (i, 0)),
            pl.BlockSpec((D, O), lambda i: (0, 0)),
            pl.BlockSpec((1, O), lambda i: (0, 0)),
        ],
        out_specs=pl.BlockSpec((blk, O), lambda i: (i, 0)),
        out_shape=jax.ShapeDtypeStruct((N, O), jnp.bfloat16),
    )(h_src, qwt, qb)


def _sc_body(nsrc_hbm, eidx_hbm, w_hbm, out_hbm, outw_hbm,
             sbuf, dbuf, wbuf, rows, bfbuf, wring, acc_sh, accw_sh,
             sem_i, sem_g, sem_s):
    core = lax.axis_index("c")
    sid = lax.axis_index("s")
    wid = core * NS + sid
    base_c = wid * CPT

    def _idx_group_copies(grp, slot):
        row0 = base_c + G * grp
        return (
            pltpu.make_async_copy(eidx_hbm.at[0, pl.ds(row0, G)],
                                  sbuf.at[pl.ds(slot * G, G)], sem_i),
            pltpu.make_async_copy(eidx_hbm.at[1, pl.ds(row0, G)],
                                  dbuf.at[pl.ds(slot * G, G)], sem_i),
            pltpu.make_async_copy(w_hbm.at[pl.ds(row0, G)],
                                  wbuf.at[pl.ds(slot * G, G)], sem_i),
        )

    def _gather(k):
        irow = k % G + G * ((k // G) % 2)
        rslot = k % 2
        return pltpu.make_async_copy(nsrc_hbm.at[sbuf.at[irow]],
                                     bfbuf.at[pl.ds(rslot * CH, CH)], sem_g)

    def _scatter(k):
        irow = k % G + G * ((k // G) % 2)
        rslot = k % 2
        return pltpu.make_async_copy(rows.at[pl.ds(rslot * CH, CH)],
                                     acc_sh.at[dbuf.at[irow]], sem_s)

    def _scatter_w(k):
        irow = k % G + G * ((k // G) % 2)
        rslot = k % 2
        return pltpu.make_async_copy(wring.at[pl.ds(rslot * CH, CH)],
                                     accw_sh.at[dbuf.at[irow]], sem_s)

    for c in _idx_group_copies(0, 0):
        c.start()

    @pl.loop(0, CH)
    def _zero(r):
        for j in range(D // 16):
            rows[CH + r, pl.ds(16 * j, 16)] = jnp.zeros((16,), jnp.float32)
        wring[CH + r, pl.ds(0, WW)] = jnp.zeros((WW,), jnp.float32)

    for c in _idx_group_copies(0, 0):
        c.wait()
    _gather(0).start()

    for b in range(ROWS_PER_TILE // CH):
        start = sid * ROWS_PER_TILE + b * CH
        pltpu.sync_copy(rows.at[pl.ds(CH, CH)], acc_sh.at[pl.ds(start, CH)])
        pltpu.sync_copy(wring.at[pl.ds(CH, CH)],
                        accw_sh.at[pl.ds(start, CH)])
    plsc.subcore_barrier()

    lane0 = lax.iota(jnp.int32, 16) == 0

    @pl.loop(0, CPT)
    def _chunk(k):
        j = k % G
        grp = k // G

        @pl.when(jnp.logical_and(j == 2, k + 3 < CPT))
        def _():
            for c in _idx_group_copies(grp + 1, (grp + 1) % 2):
                c.start()

        @pl.when(jnp.logical_and(j == G - 1, k + 1 < CPT))
        def _():
            for c in _idx_group_copies(grp + 1, (grp + 1) % 2):
                c.wait()

        @pl.when(k >= 2)
        def _():
            _scatter(k - 2).wait()
            _scatter_w(k - 2).wait()

        @pl.when(k + 1 < CPT)
        def _():
            _gather(k + 1).start()

        _gather(k).wait()

        rbase = (k % 2) * CH
        wrow = k % G + G * ((k // G) % 2)

        @pl.loop(0, CH // 16)
        def _scale(g):
            wv = wbuf[wrow, pl.ds(16 * g, 16)]
            for i in range(16):
                w = wv[i]
                r = rbase + 16 * g + i
                for u in range(D // 32):
                    b = bfbuf[r, pl.ds(32 * u, 32)]
                    f = b.astype(jnp.float32) * w
                    rows[r, pl.ds(32 * u, 32)] = f
                wring[r, pl.ds(0, WW)] = jnp.where(lane0, w, 0.0)

        _scatter(k).start(add=True)
        _scatter_w(k).start(add=True)

    for k in (CPT - 2, CPT - 1):
        _scatter(k).wait()
        _scatter_w(k).wait()
    plsc.subcore_barrier()

    start = sid * ROWS_PER_TILE
    pltpu.sync_copy(acc_sh.at[pl.ds(start, ROWS_PER_TILE)],
                    out_hbm.at[core, pl.ds(start, ROWS_PER_TILE)])
    pltpu.sync_copy(accw_sh.at[pl.ds(start, ROWS_PER_TILE)],
                    outw_hbm.at[core, pl.ds(start, ROWS_PER_TILE)])


def _sc_scatter(nsrc, eidx3, w2):
    mesh = plsc.VectorSubcoreMesh(core_axis_name="c", subcore_axis_name="s")
    kern = pl.kernel(
        _sc_body,
        out_type=(jax.ShapeDtypeStruct((NC, NP, D), jnp.float32),
                  jax.ShapeDtypeStruct((NC, NP, WW), jnp.float32)),
        mesh=mesh,
        scratch_types=[
            pltpu.VMEM((2 * G, CH), jnp.int32),
            pltpu.VMEM((2 * G, CH), jnp.int32),
            pltpu.VMEM((2 * G, CH), jnp.float32),
            pltpu.VMEM((2 * CH, D), jnp.float32),
            pltpu.VMEM((2 * CH, D), jnp.bfloat16),
            pltpu.VMEM((2 * CH, WW), jnp.float32),
            pltpu.VMEM_SHARED((NP, D), jnp.float32),
            pltpu.VMEM_SHARED((NP, WW), jnp.float32),
            pltpu.SemaphoreType.DMA,
            pltpu.SemaphoreType.DMA,
            pltpu.SemaphoreType.DMA,
        ],
        compiler_params=pltpu.CompilerParams(use_tc_tiling_on_sc=False),
    )
    return kern(nsrc, eidx3, w2)


def _out_body(acc_ref, accw_ref, hdst_ref, wwt_ref, wb_ref, o_ref):
    a = acc_ref[0] + acc_ref[1]
    aw = accw_ref[0] + accw_ref[1]
    ws = jnp.maximum(aw[:, 0:1], 1.0)
    agg = a / ws
    cat = jnp.concatenate([agg, hdst_ref[...]], axis=1)
    z = jnp.dot(cat, wwt_ref[...], preferred_element_type=jnp.float32)
    z = jnp.maximum(z + wb_ref[...], 0.0)
    nrm = jnp.sqrt(jnp.sum(z * z, axis=1, keepdims=True))
    nrm = jnp.where(nrm == 0.0, 1.0, nrm)
    o_ref[...] = z / nrm


def _finish(acc, accw, h_dst, wwt, wb):
    blk = 1000
    return pl.pallas_call(
        _out_body,
        grid=(N // blk,),
        in_specs=[
            pl.BlockSpec((NC, blk, D), lambda i: (0, i, 0)),
            pl.BlockSpec((NC, blk, WW), lambda i: (0, i, 0)),
            pl.BlockSpec((blk, D), lambda i: (i, 0)),
            pl.BlockSpec((D + O, O), lambda i: (0, 0)),
            pl.BlockSpec((1, O), lambda i: (0, 0)),
        ],
        out_specs=pl.BlockSpec((blk, O), lambda i: (i, 0)),
        out_shape=jax.ShapeDtypeStruct((N, O), jnp.float32),
    )(acc, accw, h_dst, wwt, wb)


@jax.jit
def kernel(h_src, h_dst, edge_index, edge_weight, Qw, Qb, Ww, Wb):
    eidx3 = edge_index.reshape(2, NCHUNK, CH)
    w2 = edge_weight.reshape(NCHUNK, CH)
    nsrc = _project(h_src, Qw.T, Qb.reshape(1, O))
    acc, accw = _sc_scatter(nsrc, eidx3, w2)
    return _finish(acc, accw, h_dst, Ww.T, Wb.reshape(1, O))


# speedup vs baseline: 1.7526x; 1.7526x over previous
import jax
import jax.numpy as jnp
from jax import lax
from jax.experimental import pallas as pl
from jax.experimental.pallas import tpu as pltpu
from jax.experimental.pallas import tpu_sc as plsc

N = 10000
E = 320000
D = 128
O = 128
WW = 16

NC = 2
NS = 16
NW = NC * NS
CH = 80
NCHUNK = E // CH
CPT = NCHUNK // NW
G = 5
NP = 10240
ROWS_PER_TILE = NP // NS


def _proj_body(x_ref, qwt_ref, qb_ref, o_ref):
    z = jnp.dot(x_ref[...], qwt_ref[...], preferred_element_type=jnp.float32)
    o_ref[...] = jnp.maximum(z + qb_ref[...], 0.0)


def _project(h_src, qwt, qb):
    blk = 2000
    return pl.pallas_call(
        _proj_body,
        grid=(N // blk,),
        in_specs=[
            pl.BlockSpec((blk, D), lambda i: (i, 0)),
            pl.BlockSpec((D, O), lambda i: (0, 0)),
            pl.BlockSpec((1, O), lambda i: (0, 0)),
        ],
        out_specs=pl.BlockSpec((blk, O), lambda i: (i, 0)),
        out_shape=jax.ShapeDtypeStruct((N, O), jnp.float32),
    )(h_src, qwt, qb)


def _sc_body(nsrc_hbm, eidx_hbm, w_hbm, out_hbm, outw_hbm,
             sbuf, dbuf, wbuf, rows, wring, acc_sh, accw_sh,
             sem_i, sem_g, sem_s):
    core = lax.axis_index("c")
    sid = lax.axis_index("s")
    wid = core * NS + sid
    base_c = wid * CPT

    def _idx_group_copies(grp, slot):
        row0 = base_c + G * grp
        return (
            pltpu.make_async_copy(eidx_hbm.at[0, pl.ds(row0, G)],
                                  sbuf.at[pl.ds(slot * G, G)], sem_i),
            pltpu.make_async_copy(eidx_hbm.at[1, pl.ds(row0, G)],
                                  dbuf.at[pl.ds(slot * G, G)], sem_i),
            pltpu.make_async_copy(w_hbm.at[pl.ds(row0, G)],
                                  wbuf.at[pl.ds(slot * G, G)], sem_i),
        )

    def _gather(k):
        irow = k % G + G * ((k // G) % 2)
        rslot = k % 3
        return pltpu.make_async_copy(nsrc_hbm.at[sbuf.at[irow]],
                                     rows.at[pl.ds(rslot * CH, CH)], sem_g)

    def _scatter(k):
        irow = k % G + G * ((k // G) % 2)
        rslot = k % 3
        return pltpu.make_async_copy(rows.at[pl.ds(rslot * CH, CH)],
                                     acc_sh.at[dbuf.at[irow]], sem_s)

    def _scatter_w(k):
        irow = k % G + G * ((k // G) % 2)
        rslot = k % 3
        return pltpu.make_async_copy(wring.at[pl.ds(rslot * CH, CH)],
                                     accw_sh.at[dbuf.at[irow]], sem_s)

    for c in _idx_group_copies(0, 0):
        c.start()

    @pl.loop(0, CH)
    def _zero(r):
        for j in range(D // 16):
            rows[2 * CH + r, pl.ds(16 * j, 16)] = jnp.zeros((16,), jnp.float32)
        wring[2 * CH + r, pl.ds(0, WW)] = jnp.zeros((WW,), jnp.float32)

    for c in _idx_group_copies(0, 0):
        c.wait()
    _gather(0).start()

    for b in range(ROWS_PER_TILE // CH):
        start = sid * ROWS_PER_TILE + b * CH
        pltpu.sync_copy(rows.at[pl.ds(2 * CH, CH)], acc_sh.at[pl.ds(start, CH)])
        pltpu.sync_copy(wring.at[pl.ds(2 * CH, CH)],
                        accw_sh.at[pl.ds(start, CH)])
    plsc.subcore_barrier()

    lane0 = lax.iota(jnp.int32, 16) == 0

    @pl.loop(0, CPT)
    def _chunk(k):
        j = k % G
        grp = k // G

        @pl.when(jnp.logical_and(j == 2, k + 3 < CPT))
        def _():
            for c in _idx_group_copies(grp + 1, (grp + 1) % 2):
                c.start()

        @pl.when(jnp.logical_and(j == G - 1, k + 1 < CPT))
        def _():
            for c in _idx_group_copies(grp + 1, (grp + 1) % 2):
                c.wait()

        @pl.when(k >= 2)
        def _():
            _scatter(k - 2).wait()
            _scatter_w(k - 2).wait()

        @pl.when(k + 1 < CPT)
        def _():
            _gather(k + 1).start()

        _gather(k).wait()

        rbase = (k % 3) * CH
        wrow = k % G + G * ((k // G) % 2)

        @pl.loop(0, CH // 16)
        def _scale(g):
            wv = wbuf[wrow, pl.ds(16 * g, 16)]
            for i in range(16):
                w = wv[i]
                r = rbase + 16 * g + i
                for u in range(D // 16):
                    sl = pl.ds(16 * u, 16)
                    rows[r, sl] = rows[r, sl] * w
                wring[r, pl.ds(0, WW)] = jnp.where(lane0, w, 0.0)

        _scatter(k).start(add=True)
        _scatter_w(k).start(add=True)

    for k in (CPT - 2, CPT - 1):
        _scatter(k).wait()
        _scatter_w(k).wait()
    plsc.subcore_barrier()

    start = sid * ROWS_PER_TILE
    pltpu.sync_copy(acc_sh.at[pl.ds(start, ROWS_PER_TILE)],
                    out_hbm.at[core, pl.ds(start, ROWS_PER_TILE)])
    pltpu.sync_copy(accw_sh.at[pl.ds(start, ROWS_PER_TILE)],
                    outw_hbm.at[core, pl.ds(start, ROWS_PER_TILE)])


def _sc_scatter(nsrc, eidx3, w2):
    mesh = plsc.VectorSubcoreMesh(core_axis_name="c", subcore_axis_name="s")
    kern = pl.kernel(
        _sc_body,
        out_type=(jax.ShapeDtypeStruct((NC, NP, D), jnp.float32),
                  jax.ShapeDtypeStruct((NC, NP, WW), jnp.float32)),
        mesh=mesh,
        scratch_types=[
            pltpu.VMEM((2 * G, CH), jnp.int32),
            pltpu.VMEM((2 * G, CH), jnp.int32),
            pltpu.VMEM((2 * G, CH), jnp.float32),
            pltpu.VMEM((3 * CH, D), jnp.float32),
            pltpu.VMEM((3 * CH, WW), jnp.float32),
            pltpu.VMEM_SHARED((NP, D), jnp.float32),
            pltpu.VMEM_SHARED((NP, WW), jnp.float32),
            pltpu.SemaphoreType.DMA,
            pltpu.SemaphoreType.DMA,
            pltpu.SemaphoreType.DMA,
        ],
        compiler_params=pltpu.CompilerParams(use_tc_tiling_on_sc=False),
    )
    return kern(nsrc, eidx3, w2)


def _out_body(acc_ref, accw_ref, hdst_ref, wwt_ref, wb_ref, o_ref):
    a = acc_ref[0] + acc_ref[1]
    aw = accw_ref[0] + accw_ref[1]
    ws = jnp.maximum(aw[:, 0:1], 1.0)
    agg = a / ws
    cat = jnp.concatenate([agg, hdst_ref[...]], axis=1)
    z = jnp.dot(cat, wwt_ref[...], preferred_element_type=jnp.float32)
    z = jnp.maximum(z + wb_ref[...], 0.0)
    nrm = jnp.sqrt(jnp.sum(z * z, axis=1, keepdims=True))
    nrm = jnp.where(nrm == 0.0, 1.0, nrm)
    o_ref[...] = z / nrm


def _finish(acc, accw, h_dst, wwt, wb):
    blk = 1000
    return pl.pallas_call(
        _out_body,
        grid=(N // blk,),
        in_specs=[
            pl.BlockSpec((NC, blk, D), lambda i: (0, i, 0)),
            pl.BlockSpec((NC, blk, WW), lambda i: (0, i, 0)),
            pl.BlockSpec((blk, D), lambda i: (i, 0)),
            pl.BlockSpec((D + O, O), lambda i: (0, 0)),
            pl.BlockSpec((1, O), lambda i: (0, 0)),
        ],
        out_specs=pl.BlockSpec((blk, O), lambda i: (i, 0)),
        out_shape=jax.ShapeDtypeStruct((N, O), jnp.float32),
    )(acc, accw, h_dst, wwt, wb)


@jax.jit
def kernel(h_src, h_dst, edge_index, edge_weight, Qw, Qb, Ww, Wb):
    eidx3 = edge_index.reshape(2, NCHUNK, CH)
    w2 = edge_weight.reshape(NCHUNK, CH)
    nsrc = _project(h_src, Qw.T, Qb.reshape(1, O))
    acc, accw = _sc_scatter(nsrc, eidx3, w2)
    return _finish(acc, accw, h_dst, Ww.T, Wb.reshape(1, O))
